# baseline (speedup 1.0000x reference)
import jax
import jax.numpy as jnp
from jax.experimental import pallas as pl
from jax.experimental.pallas import tpu as pltpu

_B = 2000


def _outer(x_hbm, t_hbm, w_ref, b_ref, o_hbm):
    H = b_ref.shape[1]
    T = b_ref.shape[0]
    N = x_hbm.shape[0]
    D = x_hbm.shape[1]

    def inner(x_ref, t_ref2, o_ref):
        xb = x_ref[...].astype(jnp.bfloat16)
        tb = t_ref2[...]
        p = jnp.dot(xb, w_ref[...], preferred_element_type=jnp.float32)
        acc = p[:, 0:H] + b_ref[0][None, :]
        for t in range(1, T):
            acc = jnp.where(tb == t, p[:, t * H:(t + 1) * H] + b_ref[t][None, :], acc)
        o_ref[...] = acc

    pltpu.emit_pipeline(
        inner,
        grid=(N // _B,),
        in_specs=[
            pl.BlockSpec((_B, D), lambda i: (i, 0)),
            pl.BlockSpec((_B, 1), lambda i: (i, 0)),
        ],
        out_specs=pl.BlockSpec((_B, H), lambda i: (i, 0)),
    )(x_hbm, t_hbm, o_hbm)


def kernel(x, node_types, W, b):
    N, D = x.shape
    T, H, _ = W.shape
    assert N % _B == 0
    nt2 = node_types.astype(jnp.int32).reshape(N, 1)
    Wc = jnp.swapaxes(W, 1, 2).transpose(1, 0, 2).reshape(D, T * H).astype(jnp.bfloat16)
    return pl.pallas_call(
        _outer,
        in_specs=[
            pl.BlockSpec(memory_space=pltpu.HBM),
            pl.BlockSpec(memory_space=pltpu.HBM),
            pl.BlockSpec(memory_space=pltpu.VMEM),
            pl.BlockSpec(memory_space=pltpu.VMEM),
        ],
        out_specs=pl.BlockSpec(memory_space=pltpu.HBM),
        out_shape=jax.ShapeDtypeStruct((N, H), x.dtype),
    )(x, nt2, Wc, b)
